# baseline (speedup 1.0000x reference)
import functools

import jax
import jax.numpy as jnp
from jax import lax
from jax.experimental import pallas as pl
from jax.experimental.pallas import tpu as pltpu
from jax.experimental.pallas import tpu_sc as plsc

B = 16
S = 224 * 224
D = 40
NC = 2
NS = 16
NW = NC * NS
CHUNK = S // NW
PROWS = 56


def _sc_encode_body(xt_hbm, pos_hbm, part_hbm, xv, pv, av):
    wid = lax.axis_index("s") * NC + lax.axis_index("c")
    base = wid * CHUNK
    pltpu.sync_copy(xt_hbm.at[pl.ds(base * B, CHUNK * B)], xv)
    pltpu.sync_copy(pos_hbm.at[pl.ds(base * D, CHUNK * D)], pv)

    zeros = jnp.zeros((16,), jnp.float32)

    def body(i, acc):
        xrow = xv[pl.ds(i * B, 16)]
        t = jnp.where(xrow > 0.5, 1.0, 0.0)
        p0 = pv[pl.ds(i * D, 16)]
        p1 = pv[pl.ds(i * D + 16, 16)]
        p2 = pv[pl.ds(i * D + 24, 16)]
        out = []
        for b in range(B):
            tb = t[b]
            j = 3 * b
            out += [acc[j] + tb * p0, acc[j + 1] + tb * p1,
                    acc[j + 2] + tb * p2]
        return tuple(out)

    acc = lax.fori_loop(0, CHUNK, body, (zeros,) * (3 * B), unroll=False)
    for r in range(3 * B):
        av[pl.ds(r * 16, 16)] = acc[r]

    def pbody(j, acc5):
        return tuple(acc5[k] + pv[pl.ds(j * 80 + k * 16, 16)]
                     for k in range(5))

    acc5 = lax.fori_loop(0, CHUNK * D // 80, pbody, (zeros,) * 5,
                         unroll=False)
    for k in range(5):
        av[pl.ds((3 * B + k) * 16, 16)] = acc5[k]
    for r in range(3 * B + 5, PROWS):
        av[pl.ds(r * 16, 16)] = zeros

    pltpu.sync_copy(av, part_hbm.at[wid])


def _make_sc_encode():
    mesh = plsc.VectorSubcoreMesh(core_axis_name="c", subcore_axis_name="s")
    return pl.kernel(
        _sc_encode_body,
        mesh=mesh,
        out_type=jax.ShapeDtypeStruct((NW, PROWS * 16), jnp.float32),
        scratch_types=[
            pltpu.VMEM((CHUNK * B,), jnp.float32),
            pltpu.VMEM((CHUNK * D,), jnp.float32),
            pltpu.VMEM((PROWS * 16,), jnp.float32),
        ],
    )


def _combine_kernel(pa_ref, pp_ref, vw_ref, cw_ref, out_ref):
    a48 = jnp.sum(pa_ref[...], axis=0)
    A = jnp.concatenate([a48[:, :32], a48[:, 40:48]], axis=1)
    p80 = jnp.sum(pp_ref[...], axis=0)
    p40 = p80[:, :D] + p80[:, D:]
    v0 = vw_ref[0:1, :]
    v1 = vw_ref[1:2, :]
    sample = v0 * (p40 - A) + v1 * A
    enc = jnp.where(sample > 0, 1.0, -1.0)
    out_ref[...] = lax.dot_general(
        enc, cw_ref[...], (((1,), (1,)), ((), ())),
        preferred_element_type=jnp.float32)


def kernel(x, position_weight, value_weight, classify_weight):
    x_t = jnp.transpose(x.reshape(B, S)).reshape(-1)
    pos_flat = position_weight.reshape(-1)
    part = _make_sc_encode()(x_t, pos_flat)
    part_a = part[:, :3 * B * 16].reshape(NW, B, 48)
    part_p = part[:, 3 * B * 16:(3 * B + 5) * 16].reshape(NW, 1, 80)
    return pl.pallas_call(
        _combine_kernel,
        out_shape=jax.ShapeDtypeStruct((B, classify_weight.shape[0]),
                                       jnp.float32),
    )(part_a, part_p, value_weight, classify_weight)
